# Initial kernel scaffold; baseline (speedup 1.0000x reference)
import jax
import jax.numpy as jnp
from jax.experimental import pallas as pl


def kernel(x, w_gate, W1, b1, W2, b2):
    raise NotImplementedError("write your pallas kernel here")


# speedup vs baseline: 1.0341x; 1.0341x over previous
import functools

import jax
import jax.numpy as jnp
import numpy as np
from jax import lax
from jax.experimental import pallas as pl
from jax.experimental.pallas import tpu as pltpu
from jax.experimental.pallas import tpu_sc as plsc

N = 2048
D = 1024
E = 8
K = 2
F = 2048
EPS = float(np.finfo(float).eps)

BM = 256
BF = 512
RPAD = N * K + E * BM
NBLK = RPAD // BM
NJ = F // BF

NW = 32
APW = (N * K) // NW
CHUNK = 64

_HP = jax.lax.Precision.HIGHEST


def _dot(a, b):
    return jax.lax.dot_general(a, b, (((1,), (0,)), ((), ())),
                               preferred_element_type=jnp.float32,
                               precision=_HP)


def _dot_bf16(a, b):
    return jax.lax.dot_general(a.astype(jnp.bfloat16), b.astype(jnp.bfloat16),
                               (((1,), (0,)), ((), ())),
                               preferred_element_type=jnp.float32)


def _router_body(x_ref, wg_ref, slot0_ref, slot1_ref, gates_ref, be_ref):
    logits = _dot_bf16(x_ref[...], wg_ref[...])
    lane = lax.broadcasted_iota(jnp.int32, (N, E), 1)
    m1 = jnp.max(logits, axis=1, keepdims=True)
    i1 = jnp.min(jnp.where(logits == m1, lane, E), axis=1, keepdims=True)
    masked = jnp.where(lane == i1, -jnp.inf, logits)
    m2 = jnp.max(masked, axis=1, keepdims=True)
    i2 = jnp.min(jnp.where(masked == m2, lane, E), axis=1, keepdims=True)
    t = jnp.exp(m2 - m1)
    g0 = 1.0 / (1.0 + t)
    gates_ref[...] = jnp.concatenate([g0, 1.0 - g0], axis=1)

    oh0 = (lane == i1).astype(jnp.float32)
    oh1 = (lane == i2).astype(jnp.float32)
    CH = 512
    r = lax.broadcasted_iota(jnp.int32, (CH, CH), 0)
    c = lax.broadcasted_iota(jnp.int32, (CH, CH), 1)
    tril = (r > c).astype(jnp.float32)
    base0 = jnp.zeros((1, E), jnp.float32)
    base1 = jnp.zeros((1, E), jnp.float32)
    p0, p1 = [], []
    for ci in range(N // CH):
        o0 = oh0[ci * CH:(ci + 1) * CH]
        o1 = oh1[ci * CH:(ci + 1) * CH]
        p0.append(_dot(tril, o0) + base0)
        p1.append(_dot(tril, o1) + base1)
        base0 = base0 + jnp.sum(o0, axis=0, keepdims=True)
        base1 = base1 + jnp.sum(o1, axis=0, keepdims=True)
    cnt0 = jnp.concatenate(p0, axis=0)
    cnt1 = jnp.concatenate(p1, axis=0)
    total0 = base0
    counts = base0 + base1

    pc = ((counts.astype(jnp.int32) + (BM - 1)) // BM) * BM
    er = lax.broadcasted_iota(jnp.int32, (E, E), 0)
    ec = lax.broadcasted_iota(jnp.int32, (E, E), 1)
    upT = (er < ec).astype(jnp.float32)
    offp = _dot(pc.astype(jnp.float32), upT)
    ends = offp + pc.astype(jnp.float32)

    def pick(tbl, idx):
        return jnp.sum(jnp.where(lane == idx, jnp.broadcast_to(tbl, (N, E)), 0.0),
                       axis=1, keepdims=True)

    rank0 = jnp.sum(jnp.where(lane == i1, cnt0, 0.0), axis=1, keepdims=True)
    rank1 = jnp.sum(jnp.where(lane == i2, cnt1, 0.0), axis=1, keepdims=True)
    slot0_ref[...] = (pick(offp, i1) + rank0).astype(jnp.int32)
    slot1_ref[...] = (pick(offp, i2) + pick(total0, i2) + rank1).astype(jnp.int32)

    bstart = (lax.broadcasted_iota(jnp.int32, (32, E), 0) * BM).astype(jnp.float32)
    be = jnp.sum((jnp.broadcast_to(ends, (32, E)) <= bstart).astype(jnp.int32),
                 axis=1, keepdims=True)
    be_ref[...] = jnp.clip(be, 0, E - 1)


def _router(x, w_gate):
    return pl.pallas_call(
        _router_body,
        out_shape=[
            jax.ShapeDtypeStruct((N, 1), jnp.int32),
            jax.ShapeDtypeStruct((N, 1), jnp.int32),
            jax.ShapeDtypeStruct((N, 2), jnp.float32),
            jax.ShapeDtypeStruct((32, 1), jnp.int32),
        ],
    )(x, w_gate)


def _sc_mesh():
    return plsc.VectorSubcoreMesh(core_axis_name="c", subcore_axis_name="s")


def _dispatch(x, slots_flat):
    @functools.partial(
        pl.kernel, mesh=_sc_mesh(),
        out_type=jax.ShapeDtypeStruct((RPAD, D), jnp.float32),
        scratch_types=[
            pltpu.VMEM((CHUNK,), jnp.int32),
            pltpu.VMEM((CHUNK, D), jnp.float32),
            pltpu.SemaphoreType.DMA,
        ],
    )
    def body(x_hbm, slots_hbm, xs_hbm, slot_v, rows_v, sem):
        wid = lax.axis_index("s") * 2 + lax.axis_index("c")
        base = wid * APW
        for c in range(APW // CHUNK):
            a0 = base + c * CHUNK
            n0 = lax.rem(a0, N)
            pltpu.sync_copy(slots_hbm.at[pl.ds(a0, CHUNK)], slot_v)
            pltpu.sync_copy(x_hbm.at[pl.ds(n0, CHUNK)], rows_v)
            pltpu.async_copy(rows_v, xs_hbm.at[slot_v], sem).wait()

    return body(x, slots_flat)


def _gather_back(ys, slots_flat):
    @functools.partial(
        pl.kernel, mesh=_sc_mesh(),
        out_type=jax.ShapeDtypeStruct((N * K, D), jnp.float32),
        scratch_types=[
            pltpu.VMEM((CHUNK,), jnp.int32),
            pltpu.VMEM((CHUNK, D), jnp.float32),
            pltpu.SemaphoreType.DMA,
        ],
    )
    def body(ys_hbm, slots_hbm, contrib_hbm, slot_v, rows_v, sem):
        wid = lax.axis_index("s") * 2 + lax.axis_index("c")
        base = wid * APW
        for c in range(APW // CHUNK):
            a0 = base + c * CHUNK
            pltpu.sync_copy(slots_hbm.at[pl.ds(a0, CHUNK)], slot_v)
            pltpu.async_copy(ys_hbm.at[slot_v], rows_v, sem).wait()
            pltpu.sync_copy(rows_v, contrib_hbm.at[pl.ds(a0, CHUNK)])

    return body(ys, slots_flat)


def _ffn_body(be_ref, xs_ref, w1_ref, b1_ref, w2_ref, b2_ref, out_ref, acc_ref):
    j = pl.program_id(1)

    @pl.when(j == 0)
    def _():
        acc_ref[...] = jnp.broadcast_to(b2_ref[0], (BM, D))

    h = jnp.maximum(_dot_bf16(xs_ref[...], w1_ref[0]) + b1_ref[0], 0.0)
    acc_ref[...] += _dot_bf16(h, w2_ref[0])

    @pl.when(j == NJ - 1)
    def _():
        out_ref[...] = jnp.exp(acc_ref[...])


def _ffn(be_flat, xs, W1, b1, W2, b2):
    grid_spec = pltpu.PrefetchScalarGridSpec(
        num_scalar_prefetch=1,
        grid=(NBLK, NJ),
        in_specs=[
            pl.BlockSpec((BM, D), lambda r, j, be: (r, 0)),
            pl.BlockSpec((1, D, BF), lambda r, j, be: (be[r], 0, j)),
            pl.BlockSpec((1, 1, BF), lambda r, j, be: (be[r], 0, j)),
            pl.BlockSpec((1, BF, D), lambda r, j, be: (be[r], j, 0)),
            pl.BlockSpec((1, 1, D), lambda r, j, be: (be[r], 0, 0)),
        ],
        out_specs=pl.BlockSpec((BM, D), lambda r, j, be: (r, 0)),
        scratch_shapes=[pltpu.VMEM((BM, D), jnp.float32)],
    )
    return pl.pallas_call(
        _ffn_body,
        grid_spec=grid_spec,
        out_shape=jax.ShapeDtypeStruct((RPAD, D), jnp.float32),
        compiler_params=pltpu.CompilerParams(
            dimension_semantics=("arbitrary", "arbitrary")),
    )(be_flat, xs, W1, b1.reshape(E, 1, F), W2, b2.reshape(E, 1, D))


def _combine_body(c0_ref, c1_ref, g_ref, out_ref):
    def rb(v):
        return v.astype(jnp.bfloat16).astype(jnp.float32)

    g = g_ref[...]
    s = rb(g[:, 0:1]) * rb(c0_ref[...]) + rb(g[:, 1:2]) * rb(c1_ref[...])
    out_ref[...] = jnp.log(jnp.where(s == 0.0, EPS, s))


def _combine(c0, c1, gates):
    bn = 256
    return pl.pallas_call(
        _combine_body,
        grid=(N // bn,),
        in_specs=[
            pl.BlockSpec((bn, D), lambda i: (i, 0)),
            pl.BlockSpec((bn, D), lambda i: (i, 0)),
            pl.BlockSpec((bn, 2), lambda i: (i, 0)),
        ],
        out_specs=pl.BlockSpec((bn, D), lambda i: (i, 0)),
        out_shape=jax.ShapeDtypeStruct((N, D), jnp.float32),
    )(c0, c1, gates)


def kernel(x, w_gate, W1, b1, W2, b2):
    slot0, slot1, gates, be = _router(x, w_gate)
    slots_flat = jnp.concatenate([slot0[:, 0], slot1[:, 0]], axis=0)
    be_flat = be[:, 0]
    xs = _dispatch(x, slots_flat)
    ys = _ffn(be_flat, xs, W1, b1, W2, b2)
    contrib = _gather_back(ys, slots_flat)
    return _combine(contrib[:N], contrib[N:], gates)


# speedup vs baseline: 1.4268x; 1.3797x over previous
import functools

import jax
import jax.numpy as jnp
import numpy as np
from jax import lax
from jax.experimental import pallas as pl
from jax.experimental.pallas import tpu as pltpu
from jax.experimental.pallas import tpu_sc as plsc

N = 2048
D = 1024
E = 8
K = 2
F = 2048
EPS = float(np.finfo(float).eps)

BM = 256
BF = 512
RPAD = N * K + E * BM
NBLK = RPAD // BM
NJ = F // BF

NW = 32
APW = (N * K) // NW
CHUNK = 64

_HP = jax.lax.Precision.HIGHEST


def _dot(a, b):
    return jax.lax.dot_general(a, b, (((1,), (0,)), ((), ())),
                               preferred_element_type=jnp.float32,
                               precision=_HP)


def _dot_bf16(a, b):
    return jax.lax.dot_general(a.astype(jnp.bfloat16), b.astype(jnp.bfloat16),
                               (((1,), (0,)), ((), ())),
                               preferred_element_type=jnp.float32)


def _router_body(x_ref, wg_ref, slot0_ref, slot1_ref, gates_ref, be_ref):
    logits = _dot_bf16(x_ref[...], wg_ref[...])
    lane = lax.broadcasted_iota(jnp.int32, (N, E), 1)
    m1 = jnp.max(logits, axis=1, keepdims=True)
    i1 = jnp.min(jnp.where(logits == m1, lane, E), axis=1, keepdims=True)
    masked = jnp.where(lane == i1, -jnp.inf, logits)
    m2 = jnp.max(masked, axis=1, keepdims=True)
    i2 = jnp.min(jnp.where(masked == m2, lane, E), axis=1, keepdims=True)
    t = jnp.exp(m2 - m1)
    g0 = 1.0 / (1.0 + t)
    gates_ref[...] = jnp.concatenate([g0, 1.0 - g0], axis=1)

    oh0 = (lane == i1).astype(jnp.float32)
    oh1 = (lane == i2).astype(jnp.float32)
    CH = 512
    r = lax.broadcasted_iota(jnp.int32, (CH, CH), 0)
    c = lax.broadcasted_iota(jnp.int32, (CH, CH), 1)
    tril = (r > c).astype(jnp.float32)
    base0 = jnp.zeros((1, E), jnp.float32)
    base1 = jnp.zeros((1, E), jnp.float32)
    p0, p1 = [], []
    for ci in range(N // CH):
        o0 = oh0[ci * CH:(ci + 1) * CH]
        o1 = oh1[ci * CH:(ci + 1) * CH]
        p0.append(_dot(tril, o0) + base0)
        p1.append(_dot(tril, o1) + base1)
        base0 = base0 + jnp.sum(o0, axis=0, keepdims=True)
        base1 = base1 + jnp.sum(o1, axis=0, keepdims=True)
    cnt0 = jnp.concatenate(p0, axis=0)
    cnt1 = jnp.concatenate(p1, axis=0)
    total0 = base0
    counts = base0 + base1

    pc = ((counts.astype(jnp.int32) + (BM - 1)) // BM) * BM
    er = lax.broadcasted_iota(jnp.int32, (E, E), 0)
    ec = lax.broadcasted_iota(jnp.int32, (E, E), 1)
    upT = (er < ec).astype(jnp.float32)
    offp = _dot(pc.astype(jnp.float32), upT)
    ends = offp + pc.astype(jnp.float32)

    def pick(tbl, idx):
        return jnp.sum(jnp.where(lane == idx, jnp.broadcast_to(tbl, (N, E)), 0.0),
                       axis=1, keepdims=True)

    rank0 = jnp.sum(jnp.where(lane == i1, cnt0, 0.0), axis=1, keepdims=True)
    rank1 = jnp.sum(jnp.where(lane == i2, cnt1, 0.0), axis=1, keepdims=True)
    slot0_ref[...] = (pick(offp, i1) + rank0).astype(jnp.int32)
    slot1_ref[...] = (pick(offp, i2) + pick(total0, i2) + rank1).astype(jnp.int32)

    bstart = (lax.broadcasted_iota(jnp.int32, (32, E), 0) * BM).astype(jnp.float32)
    be = jnp.sum((jnp.broadcast_to(ends, (32, E)) <= bstart).astype(jnp.int32),
                 axis=1, keepdims=True)
    be_ref[...] = jnp.clip(be, 0, E - 1)


def _router(x, w_gate):
    return pl.pallas_call(
        _router_body,
        out_shape=[
            jax.ShapeDtypeStruct((N, 1), jnp.int32),
            jax.ShapeDtypeStruct((N, 1), jnp.int32),
            jax.ShapeDtypeStruct((N, 2), jnp.float32),
            jax.ShapeDtypeStruct((32, 1), jnp.int32),
        ],
    )(x, w_gate)


def _sc_mesh():
    return plsc.VectorSubcoreMesh(core_axis_name="c", subcore_axis_name="s")


def _dispatch(x, slots_flat):
    @functools.partial(
        pl.kernel, mesh=_sc_mesh(),
        out_type=jax.ShapeDtypeStruct((RPAD, D), jnp.float32),
        scratch_types=[
            pltpu.VMEM((CHUNK,), jnp.int32),
            pltpu.VMEM((CHUNK, D), jnp.float32),
            pltpu.SemaphoreType.DMA,
        ],
    )
    def body(x_hbm, slots_hbm, xs_hbm, slot_v, rows_v, sem):
        wid = lax.axis_index("s") * 2 + lax.axis_index("c")
        base = wid * APW
        for c in range(APW // CHUNK):
            a0 = base + c * CHUNK
            n0 = lax.rem(a0, N)
            pltpu.sync_copy(slots_hbm.at[pl.ds(a0, CHUNK)], slot_v)
            pltpu.sync_copy(x_hbm.at[pl.ds(n0, CHUNK)], rows_v)
            pltpu.async_copy(rows_v, xs_hbm.at[slot_v], sem).wait()

    return body(x, slots_flat)


def _gather_back(ys, slots_flat):
    @functools.partial(
        pl.kernel, mesh=_sc_mesh(),
        out_type=jax.ShapeDtypeStruct((N * K, D), jnp.float32),
        scratch_types=[
            pltpu.VMEM((CHUNK,), jnp.int32),
            pltpu.VMEM((CHUNK, D), jnp.float32),
            pltpu.SemaphoreType.DMA,
        ],
    )
    def body(ys_hbm, slots_hbm, contrib_hbm, slot_v, rows_v, sem):
        wid = lax.axis_index("s") * 2 + lax.axis_index("c")
        base = wid * APW
        for c in range(APW // CHUNK):
            a0 = base + c * CHUNK
            pltpu.sync_copy(slots_hbm.at[pl.ds(a0, CHUNK)], slot_v)
            pltpu.async_copy(ys_hbm.at[slot_v], rows_v, sem).wait()
            pltpu.sync_copy(rows_v, contrib_hbm.at[pl.ds(a0, CHUNK)])

    return body(ys, slots_flat)


def _ffn_body(be_ref, xs_ref, w1_ref, b1_ref, w2_ref, b2_ref, out_ref):
    h = jnp.maximum(_dot_bf16(xs_ref[...], w1_ref[0]) + b1_ref[0], 0.0)
    out_ref[...] = jnp.exp(_dot_bf16(h, w2_ref[0]) + b2_ref[0])


def _ffn(be_flat, xs, W1, b1, W2, b2):
    grid_spec = pltpu.PrefetchScalarGridSpec(
        num_scalar_prefetch=1,
        grid=(NBLK,),
        in_specs=[
            pl.BlockSpec((BM, D), lambda r, be: (r, 0)),
            pl.BlockSpec((1, D, F), lambda r, be: (be[r], 0, 0)),
            pl.BlockSpec((1, 1, F), lambda r, be: (be[r], 0, 0)),
            pl.BlockSpec((1, F, D), lambda r, be: (be[r], 0, 0)),
            pl.BlockSpec((1, 1, D), lambda r, be: (be[r], 0, 0)),
        ],
        out_specs=pl.BlockSpec((BM, D), lambda r, be: (r, 0)),
    )
    return pl.pallas_call(
        _ffn_body,
        grid_spec=grid_spec,
        out_shape=jax.ShapeDtypeStruct((RPAD, D), jnp.float32),
        compiler_params=pltpu.CompilerParams(
            dimension_semantics=("arbitrary",)),
    )(be_flat, xs, W1, b1.reshape(E, 1, F), W2, b2.reshape(E, 1, D))


def _combine_body(c0_ref, c1_ref, g_ref, out_ref):
    def rb(v):
        return v.astype(jnp.bfloat16).astype(jnp.float32)

    g = g_ref[...]
    s = rb(g[:, 0:1]) * rb(c0_ref[...]) + rb(g[:, 1:2]) * rb(c1_ref[...])
    out_ref[...] = jnp.log(jnp.where(s == 0.0, EPS, s))


def _combine(c0, c1, gates):
    bn = 256
    return pl.pallas_call(
        _combine_body,
        grid=(N // bn,),
        in_specs=[
            pl.BlockSpec((bn, D), lambda i: (i, 0)),
            pl.BlockSpec((bn, D), lambda i: (i, 0)),
            pl.BlockSpec((bn, 2), lambda i: (i, 0)),
        ],
        out_specs=pl.BlockSpec((bn, D), lambda i: (i, 0)),
        out_shape=jax.ShapeDtypeStruct((N, D), jnp.float32),
    )(c0, c1, gates)


def kernel(x, w_gate, W1, b1, W2, b2):
    slot0, slot1, gates, be = _router(x, w_gate)
    slots_flat = jnp.concatenate([slot0[:, 0], slot1[:, 0]], axis=0)
    be_flat = be[:, 0]
    xs = _dispatch(x, slots_flat)
    ys = _ffn(be_flat, xs, W1, b1, W2, b2)
    contrib = _gather_back(ys, slots_flat)
    return _combine(contrib[:N], contrib[N:], gates)
